# Initial kernel scaffold; baseline (speedup 1.0000x reference)
import functools

import jax
import jax.numpy as jnp
from jax import lax
from jax.experimental import pallas as pl
from jax.experimental.pallas import tpu as pltpu
from jax.experimental.pallas import tpu_sc as plsc

B = 4096
L = 200
VOCAB = 100000
D = 128
NCLASS = 10
DP = 16

NW = 32
SPW = B // NW
IPW = SPW * L

WIN = 2
NWIN = SPW // WIN
KACC = 8

_CHUNKS = tuple(
    (s * L + o, n) for s in range(WIN) for (o, n) in ((0, 104), (104, 96))
)

VBLK = 2000


def _fold_body(w1_ref, w2_ref, e_ref, g_ref):
    m = lax.dot_general(
        w1_ref[...], w2_ref[...], (((1,), (0,)), ((), ())),
        preferred_element_type=jnp.float32, precision=lax.Precision.HIGHEST)
    g_ref[...] = lax.dot_general(
        e_ref[...], m, (((1,), (0,)), ((), ())),
        preferred_element_type=jnp.float32, precision=lax.Precision.HIGHEST)


def _fold_table(emb_table, W1, W2p):
    return pl.pallas_call(
        _fold_body,
        grid=(VOCAB // VBLK,),
        in_specs=[
            pl.BlockSpec((D, 50), lambda i: (0, 0)),
            pl.BlockSpec((50, DP), lambda i: (0, 0)),
            pl.BlockSpec((VBLK, D), lambda i: (i, 0)),
        ],
        out_specs=pl.BlockSpec((VBLK, DP), lambda i: (i, 0)),
        out_shape=jax.ShapeDtypeStruct((VOCAB, DP), jnp.float32),
    )(W1, W2p, emb_table)


def _head_body(s_ref, b1_ref, w2_ref, b2_ref, o_ref):
    c = lax.dot_general(
        b1_ref[...], w2_ref[...], (((1,), (0,)), ((), ())),
        preferred_element_type=jnp.float32, precision=lax.Precision.HIGHEST)
    o_ref[...] = jnp.tanh(s_ref[...][:, :NCLASS] + c + b2_ref[...])


def _head(S, b1, W2, b2):
    return pl.pallas_call(
        _head_body,
        out_shape=jax.ShapeDtypeStruct((B, NCLASS), jnp.float32),
    )(S, b1.reshape(1, 50), W2, b2.reshape(1, NCLASS))


def _sc_bag_sum(G, idx_flat):
    mesh = plsc.VectorSubcoreMesh(core_axis_name="c", subcore_axis_name="s")

    @functools.partial(
        pl.kernel,
        out_type=jax.ShapeDtypeStruct((B, DP), jnp.float32),
        mesh=mesh,
        scratch_types=[
            pltpu.VMEM((IPW,), jnp.int32),
            pltpu.VMEM((2, WIN * L, DP), jnp.float32),
            pltpu.VMEM((SPW, DP), jnp.float32),
            pltpu.SemaphoreType.DMA,
            pltpu.SemaphoreType.DMA,
        ],
    )
    def sc_sum(g_hbm, idx_hbm, out_hbm, idx_v, rows_v, out_v, sem_a, sem_b):
        wid = lax.axis_index("c") * 16 + lax.axis_index("s")
        base = wid * IPW

        pltpu.sync_copy(idx_hbm.at[pl.ds(base, IPW)], idx_v)

        def issue(w, buf, sem):
            woff = w * (WIN * L)
            for (o, n) in _CHUNKS:
                pltpu.async_copy(
                    g_hbm.at[idx_v.at[pl.ds(woff + o, n)]],
                    buf.at[pl.ds(o, n)], sem)

        def drain(buf, sem):
            pltpu.make_async_copy(g_hbm.at[pl.ds(0, WIN * L)], buf, sem).wait()

        def accumulate(w, buf):
            for j in range(WIN):
                roff = j * L
                accs = [buf[roff + k] for k in range(KACC)]
                for r in range(KACC, L, KACC):
                    for k in range(KACC):
                        accs[k] = accs[k] + buf[roff + r + k]
                while len(accs) > 1:
                    accs = [accs[i] + accs[i + 1]
                            for i in range(0, len(accs), 2)]
                out_v[w * WIN + j] = accs[0]

        issue(0, rows_v.at[0], sem_a)

        @pl.loop(0, NWIN, step=2)
        def _(w):
            issue(w + 1, rows_v.at[1], sem_b)
            drain(rows_v.at[0], sem_a)
            accumulate(w, rows_v.at[0])

            @pl.when(w + 2 < NWIN)
            def _():
                issue(w + 2, rows_v.at[0], sem_a)

            drain(rows_v.at[1], sem_b)
            accumulate(w + 1, rows_v.at[1])

        pltpu.sync_copy(out_v, out_hbm.at[pl.ds(wid * SPW, SPW)])

    return sc_sum(G, idx_flat)


@jax.jit
def kernel(inputs, emb_table, W1, b1, W2, b2):
    idx_flat = inputs.astype(jnp.int32).reshape(B * L)
    W2p = jnp.pad(W2, ((0, 0), (0, DP - NCLASS)))
    G = _fold_table(emb_table, W1, W2p)
    S = _sc_bag_sum(G, idx_flat)
    return _head(S, b1, W2, b2)


# speedup vs baseline: 14.2106x; 14.2106x over previous
import functools

import jax
import jax.numpy as jnp
from jax import lax
from jax.experimental import pallas as pl
from jax.experimental.pallas import tpu as pltpu
from jax.experimental.pallas import tpu_sc as plsc

B = 4096
L = 200
VOCAB = 100000
D = 128
NCLASS = 10
DP = 16

NW = 32
SPW = B // NW
IPW = SPW * L

WIN = 2
NWIN = SPW // WIN
KACC = 8

_CHUNKS = tuple(
    (s * L + o, n) for s in range(WIN) for (o, n) in ((0, 104), (104, 96))
)

VBLK = 2000


def _fold_body(w1_ref, w2_ref, e_ref, g_ref):
    m = lax.dot_general(
        w1_ref[...], w2_ref[...], (((1,), (0,)), ((), ())),
        preferred_element_type=jnp.float32, precision=lax.Precision.HIGHEST)
    g_ref[...] = lax.dot_general(
        e_ref[...], m, (((1,), (0,)), ((), ())),
        preferred_element_type=jnp.float32, precision=lax.Precision.HIGHEST)


def _fold_table(emb_table, W1, W2p):
    return pl.pallas_call(
        _fold_body,
        grid=(VOCAB // VBLK,),
        in_specs=[
            pl.BlockSpec((D, 50), lambda i: (0, 0)),
            pl.BlockSpec((50, DP), lambda i: (0, 0)),
            pl.BlockSpec((VBLK, D), lambda i: (i, 0)),
        ],
        out_specs=pl.BlockSpec((VBLK, DP), lambda i: (i, 0)),
        out_shape=jax.ShapeDtypeStruct((VOCAB, DP), jnp.float32),
    )(W1, W2p, emb_table)


def _head_body(s_ref, b1_ref, w2_ref, b2_ref, o_ref):
    c = lax.dot_general(
        b1_ref[...], w2_ref[...], (((1,), (0,)), ((), ())),
        preferred_element_type=jnp.float32, precision=lax.Precision.HIGHEST)
    o_ref[...] = jnp.tanh(s_ref[...][:, :NCLASS] + c + b2_ref[...])


def _head(S, b1, W2, b2):
    return pl.pallas_call(
        _head_body,
        out_shape=jax.ShapeDtypeStruct((B, NCLASS), jnp.float32),
    )(S, b1.reshape(1, 50), W2, b2.reshape(1, NCLASS))


def _sc_bag_sum(G, idx_flat):
    mesh = plsc.VectorSubcoreMesh(core_axis_name="c", subcore_axis_name="s")

    @functools.partial(
        pl.kernel,
        out_type=jax.ShapeDtypeStruct((B, DP), jnp.float32),
        mesh=mesh,
        scratch_types=[
            pltpu.VMEM((IPW,), jnp.int32),
            pltpu.VMEM((2, WIN * L, DP), jnp.float32),
            pltpu.VMEM((SPW, DP), jnp.float32),
            pltpu.SemaphoreType.DMA,
            pltpu.SemaphoreType.DMA,
        ],
        compiler_params=pltpu.CompilerParams(use_tc_tiling_on_sc=False),
    )
    def sc_sum(g_hbm, idx_hbm, out_hbm, idx_v, rows_v, out_v, sem_a, sem_b):
        wid = lax.axis_index("c") * 16 + lax.axis_index("s")
        base = wid * IPW

        pltpu.sync_copy(idx_hbm.at[pl.ds(base, IPW)], idx_v)

        def issue(w, buf, sem):
            woff = w * (WIN * L)
            for (o, n) in _CHUNKS:
                pltpu.async_copy(
                    g_hbm.at[idx_v.at[pl.ds(woff + o, n)]],
                    buf.at[pl.ds(o, n)], sem)

        def drain(buf, sem):
            pltpu.make_async_copy(g_hbm.at[pl.ds(0, WIN * L)], buf, sem).wait()

        def accumulate(w, buf):
            for j in range(WIN):
                roff = j * L
                accs = [buf[roff + k] for k in range(KACC)]
                for r in range(KACC, L, KACC):
                    for k in range(KACC):
                        accs[k] = accs[k] + buf[roff + r + k]
                while len(accs) > 1:
                    accs = [accs[i] + accs[i + 1]
                            for i in range(0, len(accs), 2)]
                out_v[w * WIN + j] = accs[0]

        issue(0, rows_v.at[0], sem_a)

        @pl.loop(0, NWIN, step=2)
        def _(w):
            issue(w + 1, rows_v.at[1], sem_b)
            drain(rows_v.at[0], sem_a)
            accumulate(w, rows_v.at[0])

            @pl.when(w + 2 < NWIN)
            def _():
                issue(w + 2, rows_v.at[0], sem_a)

            drain(rows_v.at[1], sem_b)
            accumulate(w + 1, rows_v.at[1])

        pltpu.sync_copy(out_v, out_hbm.at[pl.ds(wid * SPW, SPW)])

    return sc_sum(G, idx_flat)


@jax.jit
def kernel(inputs, emb_table, W1, b1, W2, b2):
    idx_flat = inputs.astype(jnp.int32).reshape(B * L)
    W2p = jnp.pad(W2, ((0, 0), (0, DP - NCLASS)))
    G = _fold_table(emb_table, W1, W2p)
    S = _sc_bag_sum(G, idx_flat)
    return _head(S, b1, W2, b2)


# speedup vs baseline: 15.4027x; 1.0839x over previous
import functools

import jax
import jax.numpy as jnp
from jax import lax
from jax.experimental import pallas as pl
from jax.experimental.pallas import tpu as pltpu
from jax.experimental.pallas import tpu_sc as plsc

B = 4096
L = 200
VOCAB = 100000
D = 128
NCLASS = 10
DP = 16

NW = 32
SPW = B // NW
IPW = SPW * L

WIN = 2
NWIN = SPW // WIN
KACC = 8

_CHUNKS = tuple(
    (s * L + o, n) for s in range(WIN) for (o, n) in ((0, 104), (104, 96))
)

VBLK = 2000


def _fold_body(w1_ref, w2_ref, e_ref, g_ref):
    m = lax.dot_general(
        w1_ref[...], w2_ref[...], (((1,), (0,)), ((), ())),
        preferred_element_type=jnp.float32, precision=lax.Precision.HIGHEST)
    g_ref[...] = lax.dot_general(
        e_ref[...], m, (((1,), (0,)), ((), ())),
        preferred_element_type=jnp.float32)


def _fold_table(emb_table, W1, W2p):
    return pl.pallas_call(
        _fold_body,
        grid=(VOCAB // VBLK,),
        in_specs=[
            pl.BlockSpec((D, 50), lambda i: (0, 0)),
            pl.BlockSpec((50, DP), lambda i: (0, 0)),
            pl.BlockSpec((VBLK, D), lambda i: (i, 0)),
        ],
        out_specs=pl.BlockSpec((VBLK, DP), lambda i: (i, 0)),
        out_shape=jax.ShapeDtypeStruct((VOCAB, DP), jnp.float32),
    )(W1, W2p, emb_table)


def _head_body(s_ref, b1_ref, w2_ref, b2_ref, o_ref):
    c = lax.dot_general(
        b1_ref[...], w2_ref[...], (((1,), (0,)), ((), ())),
        preferred_element_type=jnp.float32, precision=lax.Precision.HIGHEST)
    o_ref[...] = jnp.tanh(s_ref[...][:, :NCLASS] + c + b2_ref[...])


def _head(S, b1, W2, b2):
    return pl.pallas_call(
        _head_body,
        out_shape=jax.ShapeDtypeStruct((B, NCLASS), jnp.float32),
    )(S, b1.reshape(1, 50), W2, b2.reshape(1, NCLASS))


def _sc_bag_sum(G, idx_flat):
    mesh = plsc.VectorSubcoreMesh(core_axis_name="c", subcore_axis_name="s")

    @functools.partial(
        pl.kernel,
        out_type=jax.ShapeDtypeStruct((B, DP), jnp.float32),
        mesh=mesh,
        scratch_types=[
            pltpu.VMEM((IPW,), jnp.int32),
            pltpu.VMEM((2, WIN * L, DP), jnp.float32),
            pltpu.VMEM((SPW, DP), jnp.float32),
            pltpu.SemaphoreType.DMA,
            pltpu.SemaphoreType.DMA,
        ],
        compiler_params=pltpu.CompilerParams(use_tc_tiling_on_sc=False),
    )
    def sc_sum(g_hbm, idx_hbm, out_hbm, idx_v, rows_v, out_v, sem_a, sem_b):
        wid = lax.axis_index("c") * 16 + lax.axis_index("s")
        base = wid * IPW

        pltpu.sync_copy(idx_hbm.at[pl.ds(base, IPW)], idx_v)

        def issue(w, buf, sem):
            woff = w * (WIN * L)
            for (o, n) in _CHUNKS:
                pltpu.async_copy(
                    g_hbm.at[idx_v.at[pl.ds(woff + o, n)]],
                    buf.at[pl.ds(o, n)], sem)

        def drain(buf, sem):
            pltpu.make_async_copy(g_hbm.at[pl.ds(0, WIN * L)], buf, sem).wait()

        def accumulate(w, buf):
            for j in range(WIN):
                roff = j * L
                accs = [buf[roff + k] for k in range(KACC)]
                for r in range(KACC, L, KACC):
                    for k in range(KACC):
                        accs[k] = accs[k] + buf[roff + r + k]
                while len(accs) > 1:
                    accs = [accs[i] + accs[i + 1]
                            for i in range(0, len(accs), 2)]
                out_v[w * WIN + j] = accs[0]

        issue(0, rows_v.at[0], sem_a)

        @pl.loop(0, NWIN, step=2)
        def _(w):
            issue(w + 1, rows_v.at[1], sem_b)
            drain(rows_v.at[0], sem_a)
            accumulate(w, rows_v.at[0])

            @pl.when(w + 2 < NWIN)
            def _():
                issue(w + 2, rows_v.at[0], sem_a)

            drain(rows_v.at[1], sem_b)
            accumulate(w + 1, rows_v.at[1])

        pltpu.sync_copy(out_v, out_hbm.at[pl.ds(wid * SPW, SPW)])

    return sc_sum(G, idx_flat)


@jax.jit
def kernel(inputs, emb_table, W1, b1, W2, b2):
    idx_flat = inputs.astype(jnp.int32).reshape(B * L)
    W2p = jnp.pad(W2, ((0, 0), (0, DP - NCLASS)))
    G = _fold_table(emb_table, W1, W2p)
    S = _sc_bag_sum(G, idx_flat)
    return _head(S, b1, W2, b2)


# speedup vs baseline: 21.3364x; 1.3852x over previous
import functools

import jax
import jax.numpy as jnp
from jax import lax
from jax.experimental import pallas as pl
from jax.experimental.pallas import tpu as pltpu
from jax.experimental.pallas import tpu_sc as plsc

B = 4096
L = 200
VOCAB = 100000
D = 128
NCLASS = 10
DP = 16

NW = 32
SPW = B // NW
IPW = SPW * L

WIN = 2
NWIN = SPW // WIN
KACC = 8

_CHUNKS = tuple(
    (s * L + o, n) for s in range(WIN) for (o, n) in ((0, 104), (104, 96))
)

PACK = 128 // DP
VP = VOCAB // PACK
BP = 504


def _fold_body(w1_ref, w2_ref, e_ref, g_ref):
    m = lax.dot_general(
        w1_ref[...], w2_ref[...], (((1,), (0,)), ((), ())),
        preferred_element_type=jnp.float32, precision=lax.Precision.HIGHEST)
    e = e_ref[...]
    for j in range(PACK):
        g_ref[:, j * DP:(j + 1) * DP] = lax.dot_general(
            e[:, j, :], m, (((1,), (0,)), ((), ())),
            preferred_element_type=jnp.float32)


def _fold_table(emb_table, W1, W2p):
    e3 = emb_table.reshape(VP, PACK, D)
    gp = pl.pallas_call(
        _fold_body,
        grid=(pl.cdiv(VP, BP),),
        in_specs=[
            pl.BlockSpec((D, 50), lambda i: (0, 0)),
            pl.BlockSpec((50, DP), lambda i: (0, 0)),
            pl.BlockSpec((BP, PACK, D), lambda i: (i, 0, 0)),
        ],
        out_specs=pl.BlockSpec((BP, 128), lambda i: (i, 0)),
        out_shape=jax.ShapeDtypeStruct((VP, 128), jnp.float32),
    )(W1, W2p, e3)
    return gp.reshape(VOCAB, DP)


def _head_body(s_ref, b1_ref, w2_ref, b2_ref, o_ref):
    c = lax.dot_general(
        b1_ref[...], w2_ref[...], (((1,), (0,)), ((), ())),
        preferred_element_type=jnp.float32, precision=lax.Precision.HIGHEST)
    o_ref[...] = jnp.tanh(s_ref[...][:, :NCLASS] + c + b2_ref[...])


def _head(S, b1, W2, b2):
    return pl.pallas_call(
        _head_body,
        out_shape=jax.ShapeDtypeStruct((B, NCLASS), jnp.float32),
    )(S, b1.reshape(1, 50), W2, b2.reshape(1, NCLASS))


def _sc_bag_sum(G, idx_flat):
    mesh = plsc.VectorSubcoreMesh(core_axis_name="c", subcore_axis_name="s")

    @functools.partial(
        pl.kernel,
        out_type=jax.ShapeDtypeStruct((B, DP), jnp.float32),
        mesh=mesh,
        scratch_types=[
            pltpu.VMEM((IPW,), jnp.int32),
            pltpu.VMEM((2, WIN * L, DP), jnp.float32),
            pltpu.VMEM((SPW, DP), jnp.float32),
            pltpu.SemaphoreType.DMA,
            pltpu.SemaphoreType.DMA,
        ],
        compiler_params=pltpu.CompilerParams(use_tc_tiling_on_sc=False),
    )
    def sc_sum(g_hbm, idx_hbm, out_hbm, idx_v, rows_v, out_v, sem_a, sem_b):
        wid = lax.axis_index("c") * 16 + lax.axis_index("s")
        base = wid * IPW

        pltpu.sync_copy(idx_hbm.at[pl.ds(base, IPW)], idx_v)

        def issue(w, buf, sem):
            woff = w * (WIN * L)
            for (o, n) in _CHUNKS:
                pltpu.async_copy(
                    g_hbm.at[idx_v.at[pl.ds(woff + o, n)]],
                    buf.at[pl.ds(o, n)], sem)

        def drain(buf, sem):
            pltpu.make_async_copy(g_hbm.at[pl.ds(0, WIN * L)], buf, sem).wait()

        def accumulate(w, buf):
            for j in range(WIN):
                roff = j * L
                accs = [buf[roff + k] for k in range(KACC)]
                for r in range(KACC, L, KACC):
                    for k in range(KACC):
                        accs[k] = accs[k] + buf[roff + r + k]
                while len(accs) > 1:
                    accs = [accs[i] + accs[i + 1]
                            for i in range(0, len(accs), 2)]
                out_v[w * WIN + j] = accs[0]

        issue(0, rows_v.at[0], sem_a)

        @pl.loop(0, NWIN, step=2)
        def _(w):
            issue(w + 1, rows_v.at[1], sem_b)
            drain(rows_v.at[0], sem_a)
            accumulate(w, rows_v.at[0])

            @pl.when(w + 2 < NWIN)
            def _():
                issue(w + 2, rows_v.at[0], sem_a)

            drain(rows_v.at[1], sem_b)
            accumulate(w + 1, rows_v.at[1])

        pltpu.sync_copy(out_v, out_hbm.at[pl.ds(wid * SPW, SPW)])

    return sc_sum(G, idx_flat)


@jax.jit
def kernel(inputs, emb_table, W1, b1, W2, b2):
    idx_flat = inputs.astype(jnp.int32).reshape(B * L)
    W2p = jnp.pad(W2, ((0, 0), (0, DP - NCLASS)))
    G = _fold_table(emb_table, W1, W2p)
    S = _sc_bag_sum(G, idx_flat)
    return _head(S, b1, W2, b2)
